# baseline (speedup 1.0000x reference)
import functools

import jax
import jax.numpy as jnp
from jax import lax
from jax.experimental import pallas as pl
from jax.experimental.pallas import tpu as pltpu
from jax.experimental.pallas import tpu_sc as plsc

_NB = 2_000_000
_NA = 4_000_000
_NBT = 1000
_NAT = 2000
_NLJ = 500
_T = 200
_TP = 208

_C = 16000
_BLK = _C // 128
_NBC = _NB // _C
_NAC = _NA // _C
_NS = 2
_NW = 32


def _ff_body(btab_h, atab_h, sig_h, eps_h, bidx_h, aidx_h, ljidx_h,
             bond_o, angle_o, b_o,
             btab_v, atab_v, sig_v, eps_v, ljx_v, sg_v, esq_v, row_v,
             i0, i1, o0, o1, si0, si1, so0, so1):
    ci = lax.axis_index("c")
    si = lax.axis_index("s")
    wid = si * 2 + ci

    hb = pltpu.async_copy(btab_h, btab_v, si0)
    ha = pltpu.async_copy(atab_h, atab_v, si1)
    hb.wait()
    ha.wait()

    iota = lax.iota(jnp.int32, 16)
    zv = jnp.zeros((16,), jnp.int32)
    ov = zv + 1

    _NCC = _NBC + _NAC
    nch = (jnp.int32(_NCC) - wid + (_NW - 1)) // _NW

    def start_in(t, ibuf, isem):
        c = wid + t * _NW

        @pl.when(c < _NBC)
        def _():
            pltpu.async_copy(bidx_h.at[pl.ds(c * _C, _C)], ibuf, isem)

        @pl.when(c >= _NBC)
        def _():
            pltpu.async_copy(
                aidx_h.at[pl.ds((c - _NBC) * _C, _C)], ibuf, isem)

    def wait_in(ibuf, isem):
        pltpu.make_async_copy(bidx_h.at[pl.ds(0, _C)], ibuf, isem).wait()

    _H1 = _BLK // 2
    _H2 = _BLK - _H1

    def start_out_part(t, obuf, osem, lo, n):
        c = wid + t * _NW

        @pl.when(c < _NBC)
        def _():
            pltpu.async_copy(obuf.at[pl.ds(lo, n)],
                             bond_o.at[pl.ds(c * _BLK + lo, n)], osem)

        @pl.when(c >= _NBC)
        def _():
            pltpu.async_copy(obuf.at[pl.ds(lo, n)],
                             angle_o.at[pl.ds((c - _NBC) * _BLK + lo, n)],
                             osem)

    def wait_out(obuf, osem):
        pltpu.make_async_copy(obuf, bond_o.at[pl.ds(0, _BLK)], osem).wait()

    def compute(t, ibuf, obuf, osem):
        c = wid + t * _NW

        def gather_range(tab_v, lo, hi):
            @plsc.parallel_loop(lo, hi)
            def inner(b):
                for s in range(8):
                    iv = ibuf[pl.ds(b * 128 + s * 16, 16)]
                    k0 = plsc.load_gather(tab_v, [iv, zv])
                    rq = plsc.load_gather(tab_v, [iv, ov])
                    obuf[b, 0, pl.ds(s * 16, 16)] = k0
                    obuf[b, 1, pl.ds(s * 16, 16)] = rq

        @pl.when(c < _NBC)
        def _():
            gather_range(btab_v, 0, _H1)

        @pl.when(c >= _NBC)
        def _():
            gather_range(atab_v, 0, _H1)

        start_out_part(t, obuf, osem, 0, _H1)

        @pl.when(c < _NBC)
        def _():
            gather_range(btab_v, _H1, _BLK)

        @pl.when(c >= _NBC)
        def _():
            gather_range(atab_v, _H1, _BLK)

        start_out_part(t, obuf, osem, _H1, _H2)

    ibufs = (i0, i1)
    isems = (si0, si1)
    obufs = (o0, o1)
    osems = (so0, so1)

    for s in range(_NS):
        @pl.when(nch > s)
        def _(s=s):
            start_in(s, ibufs[s], isems[s])

    def slot(tg, t, ibuf, isem, obuf, osem):
        @pl.when(t < nch)
        def _():
            wait_in(ibuf, isem)

            @pl.when(tg > 0)
            def _():
                wait_out(obuf, osem)

            compute(t, ibuf, obuf, osem)

            @pl.when(t + _NS < nch)
            def _():
                start_in(t + _NS, ibuf, isem)

    def group(tg, carry):
        for s in range(_NS):
            slot(tg, tg * _NS + s, ibufs[s], isems[s], obufs[s], osems[s])
        return carry

    lax.fori_loop(0, (nch + _NS - 1) // _NS, group, 0)

    for s in range(_NS):
        @pl.when(nch > s)
        def _(s=s):
            wait_out(obufs[s], osems[s])

    h1 = pltpu.async_copy(sig_h, sig_v, si0)
    h2 = pltpu.async_copy(eps_h, eps_v, si1)
    ljx_v[pl.ds(192, 16)] = jnp.zeros((16,), jnp.int32)
    h3 = pltpu.async_copy(ljidx_h, ljx_v.at[pl.ds(0, _T)], so0)
    h1.wait()
    h2.wait()
    h3.wait()

    for k in range(_TP // 16):
        lj = ljx_v[pl.ds(k * 16, 16)]
        sv = plsc.load_gather(sig_v, [lj])
        ev = plsc.load_gather(eps_v, [lj])
        r = plsc.bitcast(
            jnp.int32(0x5F3759DF)
            - lax.shift_right_logical(plsc.bitcast(ev, jnp.int32), 1),
            jnp.float32)
        for _ in range(3):
            r = r * (1.5 - 0.5 * ev * r * r)
        sg_v[pl.ds(k * 16, 16)] = sv
        esq_v[pl.ds(k * 16, 16)] = ev * r

    def b_row(t, carry):
        rr = wid + t * _NW
        rv = zv + rr
        siv = plsc.load_gather(sg_v, [rv])
        eiv = plsc.load_gather(esq_v, [rv])
        for k in range(_TP // 16):
            sj = sg_v[pl.ds(k * 16, 16)]
            ej = esq_v[pl.ds(k * 16, 16)]
            s = 0.5 * (siv + sj)
            s2 = s * s
            s6 = s2 * s2 * s2
            row_v[pl.ds(k * 16, 16)] = (4.0 * s6) * (eiv * ej)
        pltpu.sync_copy(row_v.at[pl.ds(0, _T)], b_o.at[pl.ds(rr * _T, _T)])
        return carry

    nrows = jnp.where(wid < _T % _NW, _T // _NW + 1, _T // _NW)
    lax.fori_loop(0, nrows, b_row, 0)


_ff = functools.partial(
    pl.kernel,
    mesh=plsc.VectorSubcoreMesh(core_axis_name="c", subcore_axis_name="s"),
    compiler_params=pltpu.CompilerParams(
        needs_layout_passes=False, use_tc_tiling_on_sc=False),
    out_type=[
        jax.ShapeDtypeStruct((_NB // 128, 2, 128), jnp.float32),
        jax.ShapeDtypeStruct((_NA // 128, 2, 128), jnp.float32),
        jax.ShapeDtypeStruct((_T * _T,), jnp.float32),
    ],
    scratch_types=[
        pltpu.VMEM((_NBT, 2), jnp.float32),
        pltpu.VMEM((_NAT, 2), jnp.float32),
        pltpu.VMEM((_NLJ,), jnp.float32),
        pltpu.VMEM((_NLJ,), jnp.float32),
        pltpu.VMEM((_TP,), jnp.int32),
        pltpu.VMEM((_TP,), jnp.float32),
        pltpu.VMEM((_TP,), jnp.float32),
        pltpu.VMEM((_TP,), jnp.float32),
        pltpu.VMEM((_C,), jnp.int32),
        pltpu.VMEM((_C,), jnp.int32),
        pltpu.VMEM((_BLK, 2, 128), jnp.float32),
        pltpu.VMEM((_BLK, 2, 128), jnp.float32),
        pltpu.SemaphoreType.DMA,
        pltpu.SemaphoreType.DMA,
        pltpu.SemaphoreType.DMA,
        pltpu.SemaphoreType.DMA,
    ],
)
_ff_call = _ff(_ff_body)


@jax.jit
def kernel(bond_params, angle_params, lj_sigma, lj_epsilon,
           bond_idx, angle_idx, lj_idx):
    bond_p, angle_p, b = _ff_call(
        bond_params, angle_params, lj_sigma, lj_epsilon,
        bond_idx.astype(jnp.int32), angle_idx.astype(jnp.int32),
        lj_idx.astype(jnp.int32))
    return (bond_p.transpose(0, 2, 1).reshape(_NB, 2),
            angle_p.transpose(0, 2, 1).reshape(_NA, 2),
            b.reshape(_T, _T))


# speedup vs baseline: 1.0258x; 1.0258x over previous
import functools

import jax
import jax.numpy as jnp
from jax import lax
from jax.experimental import pallas as pl
from jax.experimental.pallas import tpu as pltpu
from jax.experimental.pallas import tpu_sc as plsc

_NB = 2_000_000
_NA = 4_000_000
_NBT = 1000
_NAT = 2000
_NLJ = 500
_T = 200
_TP = 208

_C = 16000
_BLK = _C // 128
_NBC = _NB // _C
_NAC = _NA // _C
_NS = 2
_NW = 32


def _ff_body(btab_h, atab_h, sig_h, eps_h, bidx_h, aidx_h, ljidx_h,
             bond_o, angle_o, b_o,
             btab_v, atab_v, sig_v, eps_v, ljx_v, sg_v, esq_v, row_v,
             i0, i1, o0, o1, si0, si1, so0, so1):
    ci = lax.axis_index("c")
    si = lax.axis_index("s")
    wid = si * 2 + ci

    hb = pltpu.async_copy(btab_h, btab_v, si0)
    ha = pltpu.async_copy(atab_h, atab_v, si1)
    hb.wait()
    ha.wait()

    iota = lax.iota(jnp.int32, 16)
    zv = jnp.zeros((16,), jnp.int32)
    ov = zv + 1

    _NCC = _NBC + _NAC
    nch = (jnp.int32(_NCC) - wid + (_NW - 1)) // _NW

    def start_in(t, ibuf, isem):
        c = wid + t * _NW

        @pl.when(c < _NBC)
        def _():
            pltpu.async_copy(bidx_h.at[pl.ds(c * _C, _C)], ibuf, isem)

        @pl.when(c >= _NBC)
        def _():
            pltpu.async_copy(
                aidx_h.at[pl.ds((c - _NBC) * _C, _C)], ibuf, isem)

    def wait_in(ibuf, isem):
        pltpu.make_async_copy(bidx_h.at[pl.ds(0, _C)], ibuf, isem).wait()

    def start_out(t, obuf, osem):
        c = wid + t * _NW

        @pl.when(c < _NBC)
        def _():
            pltpu.async_copy(obuf, bond_o.at[pl.ds(c * _BLK, _BLK)], osem)

        @pl.when(c >= _NBC)
        def _():
            pltpu.async_copy(
                obuf, angle_o.at[pl.ds((c - _NBC) * _BLK, _BLK)], osem)

    def wait_out(obuf, osem):
        pltpu.make_async_copy(obuf, bond_o.at[pl.ds(0, _BLK)], osem).wait()

    def compute(t, ibuf, obuf):
        c = wid + t * _NW

        def gather_from(tab_v):
            @plsc.parallel_loop(0, _BLK)
            def inner(b):
                for s in range(8):
                    iv = ibuf[pl.ds(b * 128 + s * 16, 16)]
                    k0 = plsc.load_gather(tab_v, [iv, zv])
                    rq = plsc.load_gather(tab_v, [iv, ov])
                    obuf[b, 0, pl.ds(s * 16, 16)] = k0
                    obuf[b, 1, pl.ds(s * 16, 16)] = rq

        @pl.when(c < _NBC)
        def _():
            gather_from(btab_v)

        @pl.when(c >= _NBC)
        def _():
            gather_from(atab_v)

    ibufs = (i0, i1)
    isems = (si0, si1)
    obufs = (o0, o1)
    osems = (so0, so1)

    for s in range(_NS):
        @pl.when(nch > s)
        def _(s=s):
            start_in(s, ibufs[s], isems[s])

    def slot(tg, t, ibuf, isem, obuf, osem):
        @pl.when(t < nch)
        def _():
            wait_in(ibuf, isem)

            @pl.when(tg > 0)
            def _():
                wait_out(obuf, osem)

            compute(t, ibuf, obuf)
            start_out(t, obuf, osem)

            @pl.when(t + _NS < nch)
            def _():
                start_in(t + _NS, ibuf, isem)

    def group(tg, carry):
        for s in range(_NS):
            slot(tg, tg * _NS + s, ibufs[s], isems[s], obufs[s], osems[s])
        return carry

    lax.fori_loop(0, (nch + _NS - 1) // _NS, group, 0)

    for s in range(_NS):
        @pl.when(nch > s)
        def _(s=s):
            wait_out(obufs[s], osems[s])

    h1 = pltpu.async_copy(sig_h, sig_v, si0)
    h2 = pltpu.async_copy(eps_h, eps_v, si1)
    ljx_v[pl.ds(192, 16)] = jnp.zeros((16,), jnp.int32)
    h3 = pltpu.async_copy(ljidx_h, ljx_v.at[pl.ds(0, _T)], so0)
    h1.wait()
    h2.wait()
    h3.wait()

    for k in range(_TP // 16):
        lj = ljx_v[pl.ds(k * 16, 16)]
        sv = plsc.load_gather(sig_v, [lj])
        ev = plsc.load_gather(eps_v, [lj])
        r = plsc.bitcast(
            jnp.int32(0x5F3759DF)
            - lax.shift_right_logical(plsc.bitcast(ev, jnp.int32), 1),
            jnp.float32)
        for _ in range(3):
            r = r * (1.5 - 0.5 * ev * r * r)
        sg_v[pl.ds(k * 16, 16)] = sv
        esq_v[pl.ds(k * 16, 16)] = ev * r

    def b_row(t, carry):
        rr = wid + t * _NW
        rv = zv + rr
        siv = plsc.load_gather(sg_v, [rv])
        eiv = plsc.load_gather(esq_v, [rv])
        for k in range(_TP // 16):
            sj = sg_v[pl.ds(k * 16, 16)]
            ej = esq_v[pl.ds(k * 16, 16)]
            s = 0.5 * (siv + sj)
            s2 = s * s
            s6 = s2 * s2 * s2
            row_v[pl.ds(k * 16, 16)] = (4.0 * s6) * (eiv * ej)
        pltpu.sync_copy(row_v.at[pl.ds(0, _T)], b_o.at[pl.ds(rr * _T, _T)])
        return carry

    nrows = jnp.where(wid < _T % _NW, _T // _NW + 1, _T // _NW)
    lax.fori_loop(0, nrows, b_row, 0)


_ff = functools.partial(
    pl.kernel,
    mesh=plsc.VectorSubcoreMesh(core_axis_name="c", subcore_axis_name="s"),
    compiler_params=pltpu.CompilerParams(
        needs_layout_passes=False, use_tc_tiling_on_sc=False),
    out_type=[
        jax.ShapeDtypeStruct((_NB // 128, 2, 128), jnp.float32),
        jax.ShapeDtypeStruct((_NA // 128, 2, 128), jnp.float32),
        jax.ShapeDtypeStruct((_T * _T,), jnp.float32),
    ],
    scratch_types=[
        pltpu.VMEM((_NBT, 2), jnp.float32),
        pltpu.VMEM((_NAT, 2), jnp.float32),
        pltpu.VMEM((_NLJ,), jnp.float32),
        pltpu.VMEM((_NLJ,), jnp.float32),
        pltpu.VMEM((_TP,), jnp.int32),
        pltpu.VMEM((_TP,), jnp.float32),
        pltpu.VMEM((_TP,), jnp.float32),
        pltpu.VMEM((_TP,), jnp.float32),
        pltpu.VMEM((_C,), jnp.int32),
        pltpu.VMEM((_C,), jnp.int32),
        pltpu.VMEM((_BLK, 2, 128), jnp.float32),
        pltpu.VMEM((_BLK, 2, 128), jnp.float32),
        pltpu.SemaphoreType.DMA,
        pltpu.SemaphoreType.DMA,
        pltpu.SemaphoreType.DMA,
        pltpu.SemaphoreType.DMA,
    ],
)
_ff_call = _ff(_ff_body)


@jax.jit
def kernel(bond_params, angle_params, lj_sigma, lj_epsilon,
           bond_idx, angle_idx, lj_idx):
    bond_p, angle_p, b = _ff_call(
        bond_params, angle_params, lj_sigma, lj_epsilon,
        bond_idx.astype(jnp.int32), angle_idx.astype(jnp.int32),
        lj_idx.astype(jnp.int32))
    return (bond_p.transpose(0, 2, 1).reshape(_NB, 2),
            angle_p.transpose(0, 2, 1).reshape(_NA, 2),
            b.reshape(_T, _T))
